# baseline (speedup 1.0000x reference)
import dataclasses
import functools

import jax
import jax.numpy as jnp
from jax import lax
from jax.experimental import pallas as pl
from jax.experimental.pallas import tpu as pltpu
from jax.experimental.pallas import tpu_sc as plsc

_NC = 2
_NS = 16
_L = 16
_C = 6400


@functools.partial(jax.jit, static_argnames=("n_edges", "d", "n_nodes"))
def _sc_edge_dot(ht, src, dst, *, n_edges, d, n_nodes):
    per_sc = n_edges // _NC
    n_chunks = per_sc // _C
    npairs = (n_chunks - 1) // 2
    assert n_chunks == 2 * npairs + 1
    nf = d // _NS
    sub = _C // _NS
    spg = sub // _L
    blk = _NS * sub
    assert sub % _L == 0 and sub % 8 == 0

    mesh = plsc.VectorSubcoreMesh(core_axis_name="c", subcore_axis_name="s")
    cp = pltpu.CompilerParams()
    if "needs_layout_passes" in pltpu.CompilerParams.__dataclass_fields__:
        cp = dataclasses.replace(cp, needs_layout_passes=False)

    @functools.partial(
        pl.kernel,
        compiler_params=cp,
        out_type=[
            jax.ShapeDtypeStruct((n_edges,), jnp.float32),
            jax.ShapeDtypeStruct((2 * _NC * _NS * blk,), jnp.float32),
        ],
        mesh=mesh,
        scratch_types=[
            pltpu.VMEM((nf, n_nodes), jnp.float32),
            pltpu.VMEM((_C,), jnp.int32),
            pltpu.VMEM((_C,), jnp.int32),
            pltpu.VMEM((_C,), jnp.int32),
            pltpu.VMEM((_C,), jnp.int32),
            pltpu.VMEM((blk,), jnp.float32),
            pltpu.VMEM((_NS * sub,), jnp.float32),
            pltpu.VMEM((sub,), jnp.float32),
            pltpu.SemaphoreType.DMA,
            pltpu.SemaphoreType.DMA,
            pltpu.SemaphoreType.DMA,
            pltpu.SemaphoreType.DMA,
            pltpu.SemaphoreType.DMA,
            pltpu.SemaphoreType.DMA,
        ],
    )
    def k(ht_hbm, src_hbm, dst_hbm, out_hbm, ex_hbm,
          hsl, si0, di0, si1, di1, part_v, red_v, outb_v,
          hs_sem, is0, id0, is1, id1, ex_sem):
        cid = lax.axis_index("c")
        tid = lax.axis_index("s")
        base_sc = cid * per_sc

        cph = pltpu.make_async_copy(
            ht_hbm.at[pl.ds(tid * nf, nf)], hsl, hs_sem)
        cph.start()

        def idx_start(c, si, di, ssem, dsem):
            pltpu.make_async_copy(
                src_hbm.at[pl.ds(base_sc + c * _C, _C)], si, ssem).start()
            pltpu.make_async_copy(
                dst_hbm.at[pl.ds(base_sc + c * _C, _C)], di, dsem).start()

        def idx_wait(c, si, di, ssem, dsem):
            pltpu.make_async_copy(
                src_hbm.at[pl.ds(base_sc + c * _C, _C)], si, ssem).wait()
            pltpu.make_async_copy(
                dst_hbm.at[pl.ds(base_sc + c * _C, _C)], di, dsem).wait()

        idx_start(0, si0, di0, is0, id0)
        cph.wait()

        def body(c, slot, si, di):
            @plsc.parallel_loop(0, _NS, 1, unroll=2)
            def _(p):
                for j in range(spg):
                    off = p * sub + j * _L
                    s16 = si[pl.ds(off, _L)]
                    d16 = di[pl.ds(off, _L)]
                    acc0 = jnp.zeros((_L,), jnp.float32)
                    acc1 = jnp.zeros((_L,), jnp.float32)
                    for f in range(nf):
                        row = jnp.full((_L,), f, jnp.int32)
                        prod = (plsc.load_gather(hsl, [row, s16])
                                * plsc.load_gather(hsl, [row, d16]))
                        if f % 2 == 0:
                            acc0 = acc0 + prod
                        else:
                            acc1 = acc1 + prod
                    part_v[pl.ds(off, _L)] = acc0 + acc1

            sbase = (slot * _NC + cid) * _NS * blk
            pltpu.sync_copy(part_v, ex_hbm.at[pl.ds(sbase + tid * blk, blk)])
            plsc.subcore_barrier()
            for w in range(_NS):
                pltpu.make_async_copy(
                    ex_hbm.at[pl.ds(sbase + w * blk + tid * sub, sub)],
                    red_v.at[pl.ds(w * sub, sub)], ex_sem).start()
            for w in range(_NS):
                pltpu.make_async_copy(
                    ex_hbm.at[pl.ds(sbase + w * blk + tid * sub, sub)],
                    red_v.at[pl.ds(w * sub, sub)], ex_sem).wait()

            @plsc.parallel_loop(0, spg, 1, unroll=2)
            def _(j):
                accs = [red_v[pl.ds(r * sub + j * _L, _L)] for r in range(4)]
                for r in range(4, _NS):
                    accs[r % 4] = accs[r % 4] + red_v[
                        pl.ds(r * sub + j * _L, _L)]
                outb_v[pl.ds(j * _L, _L)] = (
                    (accs[0] + accs[1]) + (accs[2] + accs[3]))

            pltpu.sync_copy(
                outb_v,
                out_hbm.at[pl.ds(base_sc + c * _C + tid * sub, sub)])

        @pl.loop(0, npairs)
        def _(i):
            c0 = 2 * i
            idx_start(c0 + 1, si1, di1, is1, id1)
            idx_wait(c0, si0, di0, is0, id0)
            body(c0, 0, si0, di0)
            idx_start(c0 + 2, si0, di0, is0, id0)
            idx_wait(c0 + 1, si1, di1, is1, id1)
            body(c0 + 1, 1, si1, di1)

        idx_wait(n_chunks - 1, si0, di0, is0, id0)
        body(n_chunks - 1, 0, si0, di0)

    return k(ht, src, dst)


def kernel(h, edge_index):
    n_nodes, d = h.shape
    n_edges = edge_index.shape[1]
    assert n_edges % (_NC * _C) == 0 and d % _NS == 0
    ht = h.T
    src = edge_index[0].astype(jnp.int32)
    dst = edge_index[1].astype(jnp.int32)
    score, _ = _sc_edge_dot(ht, src, dst,
                            n_edges=n_edges, d=d, n_nodes=n_nodes)
    return score.reshape(n_edges, 1)


# speedup vs baseline: 1.6116x; 1.6116x over previous
import dataclasses
import functools

import jax
import jax.numpy as jnp
from jax import lax
from jax.experimental import pallas as pl
from jax.experimental.pallas import tpu as pltpu
from jax.experimental.pallas import tpu_sc as plsc

_NC = 2
_NS = 16
_L = 16
_C = 1280


@functools.partial(jax.jit, static_argnames=("n_edges", "d", "n_nodes"))
def _sc_edge_dot(ht, src, dst, *, n_edges, d, n_nodes):
    per_sc = n_edges // _NC
    n_chunks = per_sc // _C
    npairs = (n_chunks - 1) // 2
    assert n_chunks == 2 * npairs + 1
    nf = d // _NS
    nq = nf // 2
    sub = _C // _NS
    spg = sub // _L
    assert sub % _L == 0

    mesh = plsc.VectorSubcoreMesh(core_axis_name="c", subcore_axis_name="s")
    cp = pltpu.CompilerParams()
    if "needs_layout_passes" in pltpu.CompilerParams.__dataclass_fields__:
        cp = dataclasses.replace(cp, needs_layout_passes=False)

    @functools.partial(
        pl.kernel,
        compiler_params=cp,
        out_type=jax.ShapeDtypeStruct((n_edges,), jnp.float32),
        mesh=mesh,
        scratch_types=[
            pltpu.VMEM((nq, n_nodes), jnp.int32),
            pltpu.VMEM((_C,), jnp.int32),
            pltpu.VMEM((_C,), jnp.int32),
            pltpu.VMEM((_C,), jnp.int32),
            pltpu.VMEM((_C,), jnp.int32),
            pltpu.VMEM((_NS, 128), jnp.float32),
            pltpu.VMEM((_NS, 128), jnp.float32),
            pltpu.VMEM((sub,), jnp.float32),
            pltpu.VMEM_SHARED((2, _NS, _NS, 128), jnp.float32),
            pltpu.SemaphoreType.DMA,
            pltpu.SemaphoreType.DMA,
            pltpu.SemaphoreType.DMA,
            pltpu.SemaphoreType.DMA,
            pltpu.SemaphoreType.DMA,
        ],
    )
    def k(ht_hbm, src_hbm, dst_hbm, out_hbm,
          hsl, si0, di0, si1, di1, part_v, red_v, outb_v, ex_sh,
          hs_sem, is0, id0, is1, id1):
        cid = lax.axis_index("c")
        tid = lax.axis_index("s")
        base_sc = cid * per_sc

        cph = pltpu.make_async_copy(
            ht_hbm.at[pl.ds(tid * nq, nq)], hsl, hs_sem)
        cph.start()

        def idx_start(c, si, di, ssem, dsem):
            pltpu.make_async_copy(
                src_hbm.at[pl.ds(base_sc + c * _C, _C)], si, ssem).start()
            pltpu.make_async_copy(
                dst_hbm.at[pl.ds(base_sc + c * _C, _C)], di, dsem).start()

        def idx_wait(c, si, di, ssem, dsem):
            pltpu.make_async_copy(
                src_hbm.at[pl.ds(base_sc + c * _C, _C)], si, ssem).wait()
            pltpu.make_async_copy(
                dst_hbm.at[pl.ds(base_sc + c * _C, _C)], di, dsem).wait()

        idx_start(0, si0, di0, is0, id0)
        cph.wait()

        def body(c, slot, si, di):
            @pl.loop(0, _NS)
            def _(p):
                for j in range(spg):
                    off = p * sub + j * _L
                    s16 = si[pl.ds(off, _L)]
                    d16 = di[pl.ds(off, _L)]
                    acc0 = jnp.zeros((_L,), jnp.float32)
                    acc1 = jnp.zeros((_L,), jnp.float32)
                    for q in range(nq):
                        row = jnp.full((_L,), q, jnp.int32)
                        ws = plsc.load_gather(hsl, [row, s16])
                        wd = plsc.load_gather(hsl, [row, d16])
                        sa, sb = plsc.unpack(
                            plsc.bitcast(ws, jnp.bfloat16),
                            format=plsc.PackFormat.INTERLEAVED,
                            preferred_element_type=jnp.float32)
                        da, db = plsc.unpack(
                            plsc.bitcast(wd, jnp.bfloat16),
                            format=plsc.PackFormat.INTERLEAVED,
                            preferred_element_type=jnp.float32)
                        acc0 = acc0 + sa * da
                        acc1 = acc1 + sb * db
                    part_v[p, pl.ds(j * _L, _L)] = acc0 + acc1

            pltpu.sync_copy(part_v, ex_sh.at[slot, tid])
            plsc.subcore_barrier()
            pltpu.sync_copy(ex_sh.at[slot, :, tid], red_v)

            for j in range(spg):
                acc = red_v[0, pl.ds(j * _L, _L)]
                for r in range(1, _NS):
                    acc = acc + red_v[r, pl.ds(j * _L, _L)]
                outb_v[pl.ds(j * _L, _L)] = acc

            pltpu.sync_copy(
                outb_v,
                out_hbm.at[pl.ds(base_sc + c * _C + tid * sub, sub)])

        @pl.loop(0, npairs)
        def _(i):
            c0 = 2 * i
            idx_start(c0 + 1, si1, di1, is1, id1)
            idx_wait(c0, si0, di0, is0, id0)
            body(c0, 0, si0, di0)
            idx_start(c0 + 2, si0, di0, is0, id0)
            idx_wait(c0 + 1, si1, di1, is1, id1)
            body(c0 + 1, 1, si1, di1)

        idx_wait(n_chunks - 1, si0, di0, is0, id0)
        body(n_chunks - 1, 0, si0, di0)

    return k(ht, src, dst)


def kernel(h, edge_index):
    n_nodes, d = h.shape
    n_edges = edge_index.shape[1]
    assert n_edges % (_NC * _C) == 0 and d % (2 * _NS) == 0
    hb = h.astype(jnp.bfloat16)
    ht = jax.lax.bitcast_convert_type(
        hb.reshape(n_nodes, d // 2, 2), jnp.int32).T
    src = edge_index[0].astype(jnp.int32)
    dst = edge_index[1].astype(jnp.int32)
    score = _sc_edge_dot(ht, src, dst, n_edges=n_edges, d=d, n_nodes=n_nodes)
    return score.reshape(n_edges, 1)


# speedup vs baseline: 1.7201x; 1.0673x over previous
import dataclasses
import functools

import jax
import jax.numpy as jnp
from jax import lax
from jax.experimental import pallas as pl
from jax.experimental.pallas import tpu as pltpu
from jax.experimental.pallas import tpu_sc as plsc

_NC = 2
_NS = 16
_L = 16
_C = 1280


@functools.partial(jax.jit, static_argnames=("n_edges", "d", "n_nodes"))
def _sc_edge_dot(ht, src, dst, *, n_edges, d, n_nodes):
    per_sc = n_edges // _NC
    n_chunks = per_sc // _C
    npairs = (n_chunks - 1) // 2
    assert n_chunks == 2 * npairs + 1 and n_chunks >= 3
    nf = d // _NS
    nq = nf // 2
    sub = _C // _NS
    spg = sub // _L
    assert sub % _L == 0

    mesh = plsc.VectorSubcoreMesh(core_axis_name="c", subcore_axis_name="s")
    cp = pltpu.CompilerParams()
    if "needs_layout_passes" in pltpu.CompilerParams.__dataclass_fields__:
        cp = dataclasses.replace(cp, needs_layout_passes=False)

    @functools.partial(
        pl.kernel,
        compiler_params=cp,
        out_type=jax.ShapeDtypeStruct((n_edges,), jnp.float32),
        mesh=mesh,
        scratch_types=[
            pltpu.VMEM((nq, n_nodes), jnp.int32),
            pltpu.VMEM((_C,), jnp.int32),
            pltpu.VMEM((_C,), jnp.int32),
            pltpu.VMEM((_C,), jnp.int32),
            pltpu.VMEM((_C,), jnp.int32),
            pltpu.VMEM((_NS, 128), jnp.float32),
            pltpu.VMEM((_NS, 128), jnp.float32),
            pltpu.VMEM((_NS, 128), jnp.float32),
            pltpu.VMEM((sub,), jnp.float32),
            pltpu.VMEM((sub,), jnp.float32),
            pltpu.VMEM_SHARED((2, _NS, _NS, 128), jnp.float32),
            pltpu.SemaphoreType.DMA,
            pltpu.SemaphoreType.DMA,
            pltpu.SemaphoreType.DMA,
            pltpu.SemaphoreType.DMA,
            pltpu.SemaphoreType.DMA,
            pltpu.SemaphoreType.DMA,
            pltpu.SemaphoreType.DMA,
            pltpu.SemaphoreType.DMA,
            pltpu.SemaphoreType.DMA,
        ],
    )
    def k(ht_hbm, src_hbm, dst_hbm, out_hbm,
          hsl, si0, di0, si1, di1, part_v, red0, red1, ob0, ob1, ex_sh,
          hs_sem, is0, id0, is1, id1, rs0, rs1, os0, os1):
        cid = lax.axis_index("c")
        tid = lax.axis_index("s")
        base_sc = cid * per_sc

        cph = pltpu.make_async_copy(
            ht_hbm.at[pl.ds(tid * nq, nq)], hsl, hs_sem)
        cph.start()

        def idx_start(c, si, di, ssem, dsem):
            pltpu.make_async_copy(
                src_hbm.at[pl.ds(base_sc + c * _C, _C)], si, ssem).start()
            pltpu.make_async_copy(
                dst_hbm.at[pl.ds(base_sc + c * _C, _C)], di, dsem).start()

        def idx_wait(c, si, di, ssem, dsem):
            pltpu.make_async_copy(
                src_hbm.at[pl.ds(base_sc + c * _C, _C)], si, ssem).wait()
            pltpu.make_async_copy(
                dst_hbm.at[pl.ds(base_sc + c * _C, _C)], di, dsem).wait()

        def out_ref(c):
            return out_hbm.at[pl.ds(base_sc + c * _C + tid * sub, sub)]

        def phase_a(si, di):
            @pl.loop(0, _NS)
            def _(p):
                for j in range(spg):
                    off = p * sub + j * _L
                    s16 = si[pl.ds(off, _L)]
                    d16 = di[pl.ds(off, _L)]
                    acc0 = jnp.zeros((_L,), jnp.float32)
                    acc1 = jnp.zeros((_L,), jnp.float32)
                    for q in range(nq):
                        row = jnp.full((_L,), q, jnp.int32)
                        ws = plsc.load_gather(hsl, [row, s16])
                        wd = plsc.load_gather(hsl, [row, d16])
                        sa, sb = plsc.unpack(
                            plsc.bitcast(ws, jnp.bfloat16),
                            format=plsc.PackFormat.INTERLEAVED,
                            preferred_element_type=jnp.float32)
                        da, db = plsc.unpack(
                            plsc.bitcast(wd, jnp.bfloat16),
                            format=plsc.PackFormat.INTERLEAVED,
                            preferred_element_type=jnp.float32)
                        acc0 = acc0 + sa * da
                        acc1 = acc1 + sb * db
                    part_v[p, pl.ds(j * _L, _L)] = acc0 + acc1

        def phase_w(slot):
            pltpu.sync_copy(part_v, ex_sh.at[slot, tid])
            plsc.subcore_barrier()

        def red_start(slot, red, rsem):
            pltpu.make_async_copy(ex_sh.at[slot, :, tid], red, rsem).start()

        def phase_r(c, slot, red, rsem, outb, osem):
            pltpu.make_async_copy(ex_sh.at[slot, :, tid], red, rsem).wait()
            plsc.subcore_barrier()

            @pl.when(c >= 2)
            def _():
                pltpu.make_async_copy(outb, out_ref(c), osem).wait()

            @pl.loop(0, spg)
            def _(j):
                accs = [red[r, pl.ds(j * _L, _L)] for r in range(4)]
                for r in range(4, _NS):
                    accs[r % 4] = accs[r % 4] + red[r, pl.ds(j * _L, _L)]
                outb[pl.ds(j * _L, _L)] = (
                    (accs[0] + accs[1]) + (accs[2] + accs[3]))

            pltpu.make_async_copy(outb, out_ref(c), osem).start()

        idx_start(0, si0, di0, is0, id0)
        idx_start(1, si1, di1, is1, id1)
        cph.wait()

        idx_wait(0, si0, di0, is0, id0)
        phase_a(si0, di0)
        idx_start(2, si0, di0, is0, id0)
        phase_w(0)
        red_start(0, red0, rs0)

        @pl.loop(0, npairs)
        def _(i):
            c0 = 2 * i
            idx_wait(c0 + 1, si1, di1, is1, id1)
            phase_a(si1, di1)

            @pl.when(c0 + 3 < n_chunks)
            def _():
                idx_start(c0 + 3, si1, di1, is1, id1)

            phase_w(1)
            red_start(1, red1, rs1)
            phase_r(c0, 0, red0, rs0, ob0, os0)

            idx_wait(c0 + 2, si0, di0, is0, id0)
            phase_a(si0, di0)

            @pl.when(c0 + 4 < n_chunks)
            def _():
                idx_start(c0 + 4, si0, di0, is0, id0)

            phase_w(0)
            red_start(0, red0, rs0)
            phase_r(c0 + 1, 1, red1, rs1, ob1, os1)

        phase_r(n_chunks - 1, 0, red0, rs0, ob0, os0)
        pltpu.make_async_copy(ob1, out_ref(n_chunks - 2), os1).wait()
        pltpu.make_async_copy(ob0, out_ref(n_chunks - 1), os0).wait()

    return k(ht, src, dst)


def kernel(h, edge_index):
    n_nodes, d = h.shape
    n_edges = edge_index.shape[1]
    assert n_edges % (_NC * _C) == 0 and d % (2 * _NS) == 0
    hb = h.astype(jnp.bfloat16)
    ht = jax.lax.bitcast_convert_type(
        hb.reshape(n_nodes, d // 2, 2), jnp.int32).T
    src = edge_index[0].astype(jnp.int32)
    dst = edge_index[1].astype(jnp.int32)
    score = _sc_edge_dot(ht, src, dst, n_edges=n_edges, d=d, n_nodes=n_nodes)
    return score.reshape(n_edges, 1)


# speedup vs baseline: 1.8135x; 1.0543x over previous
import dataclasses
import functools

import jax
import jax.numpy as jnp
from jax import lax
from jax.experimental import pallas as pl
from jax.experimental.pallas import tpu as pltpu
from jax.experimental.pallas import tpu_sc as plsc

_NC = 2
_NS = 16
_L = 16
_C = 1280


@functools.partial(jax.jit, static_argnames=("n_edges", "d", "n_nodes"))
def _sc_edge_dot(ht, src, dst, *, n_edges, d, n_nodes):
    per_sc = n_edges // _NC
    n_chunks = per_sc // _C
    npairs = (n_chunks - 1) // 2
    assert n_chunks == 2 * npairs + 1 and n_chunks >= 3
    nf = d // _NS
    nq = nf // 2
    sub = _C // _NS
    spg = sub // _L
    assert sub % _L == 0

    mesh = plsc.VectorSubcoreMesh(core_axis_name="c", subcore_axis_name="s")
    cp = pltpu.CompilerParams()
    if "needs_layout_passes" in pltpu.CompilerParams.__dataclass_fields__:
        cp = dataclasses.replace(cp, needs_layout_passes=False)

    @functools.partial(
        pl.kernel,
        compiler_params=cp,
        out_type=jax.ShapeDtypeStruct((n_edges,), jnp.float32),
        mesh=mesh,
        scratch_types=[
            pltpu.VMEM((nq, n_nodes), jnp.int32),
            pltpu.VMEM((_C,), jnp.int32),
            pltpu.VMEM((_C,), jnp.int32),
            pltpu.VMEM((_C,), jnp.int32),
            pltpu.VMEM((_C,), jnp.int32),
            pltpu.VMEM((_NS, 128), jnp.float32),
            pltpu.VMEM((_NS, 128), jnp.float32),
            pltpu.VMEM((_NS, 128), jnp.float32),
            pltpu.VMEM((sub,), jnp.float32),
            pltpu.VMEM((sub,), jnp.float32),
            pltpu.VMEM_SHARED((2, _NS, _NS, 128), jnp.float32),
            pltpu.SemaphoreType.DMA,
            pltpu.SemaphoreType.DMA,
            pltpu.SemaphoreType.DMA,
            pltpu.SemaphoreType.DMA,
            pltpu.SemaphoreType.DMA,
            pltpu.SemaphoreType.DMA,
            pltpu.SemaphoreType.DMA,
            pltpu.SemaphoreType.DMA,
            pltpu.SemaphoreType.DMA,
        ],
    )
    def k(ht_hbm, src_hbm, dst_hbm, out_hbm,
          hsl, si0, di0, si1, di1, part_v, red0, red1, ob0, ob1, ex_sh,
          hs_sem, is0, id0, is1, id1, rs0, rs1, os0, os1):
        cid = lax.axis_index("c")
        tid = lax.axis_index("s")
        base_sc = cid * per_sc

        cph = pltpu.make_async_copy(
            ht_hbm.at[pl.ds(tid * nq, nq)], hsl, hs_sem)
        cph.start()

        def idx_start(c, si, di, ssem, dsem):
            pltpu.make_async_copy(
                src_hbm.at[pl.ds(base_sc + c * _C, _C)], si, ssem).start()
            pltpu.make_async_copy(
                dst_hbm.at[pl.ds(base_sc + c * _C, _C)], di, dsem).start()

        def idx_wait(c, si, di, ssem, dsem):
            pltpu.make_async_copy(
                src_hbm.at[pl.ds(base_sc + c * _C, _C)], si, ssem).wait()
            pltpu.make_async_copy(
                dst_hbm.at[pl.ds(base_sc + c * _C, _C)], di, dsem).wait()

        def out_ref(c):
            return out_hbm.at[pl.ds(base_sc + c * _C + tid * sub, sub)]

        def phase_a(si, di):
            @pl.loop(0, _NS)
            def _(p):
                for j in range(spg):
                    off = p * sub + j * _L
                    s16 = si[pl.ds(off, _L)]
                    d16 = di[pl.ds(off, _L)]
                    acc0 = jnp.zeros((_L,), jnp.float32)
                    acc1 = jnp.zeros((_L,), jnp.float32)
                    for q in range(nq):
                        row = jnp.full((_L,), q, jnp.int32)
                        ws = plsc.load_gather(hsl, [row, s16])
                        wd = plsc.load_gather(hsl, [row, d16])
                        pm = (plsc.bitcast(ws, jnp.bfloat16)
                              * plsc.bitcast(wd, jnp.bfloat16))
                        pa, pb = plsc.unpack(
                            pm,
                            format=plsc.PackFormat.INTERLEAVED,
                            preferred_element_type=jnp.float32)
                        acc0 = acc0 + pa
                        acc1 = acc1 + pb
                    part_v[p, pl.ds(j * _L, _L)] = acc0 + acc1

        def phase_w(slot):
            pltpu.sync_copy(part_v, ex_sh.at[slot, tid])
            plsc.subcore_barrier()

        def red_start(slot, red, rsem):
            pltpu.make_async_copy(ex_sh.at[slot, :, tid], red, rsem).start()

        def phase_r(c, slot, red, rsem, outb, osem):
            pltpu.make_async_copy(ex_sh.at[slot, :, tid], red, rsem).wait()
            plsc.subcore_barrier()

            @pl.when(c >= 2)
            def _():
                pltpu.make_async_copy(outb, out_ref(c), osem).wait()

            @pl.loop(0, spg)
            def _(j):
                accs = [red[r, pl.ds(j * _L, _L)] for r in range(4)]
                for r in range(4, _NS):
                    accs[r % 4] = accs[r % 4] + red[r, pl.ds(j * _L, _L)]
                outb[pl.ds(j * _L, _L)] = (
                    (accs[0] + accs[1]) + (accs[2] + accs[3]))

            pltpu.make_async_copy(outb, out_ref(c), osem).start()

        idx_start(0, si0, di0, is0, id0)
        idx_start(1, si1, di1, is1, id1)
        cph.wait()

        idx_wait(0, si0, di0, is0, id0)
        phase_a(si0, di0)
        idx_start(2, si0, di0, is0, id0)
        phase_w(0)
        red_start(0, red0, rs0)

        @pl.loop(0, npairs)
        def _(i):
            c0 = 2 * i
            idx_wait(c0 + 1, si1, di1, is1, id1)
            phase_a(si1, di1)

            @pl.when(c0 + 3 < n_chunks)
            def _():
                idx_start(c0 + 3, si1, di1, is1, id1)

            phase_w(1)
            red_start(1, red1, rs1)
            phase_r(c0, 0, red0, rs0, ob0, os0)

            idx_wait(c0 + 2, si0, di0, is0, id0)
            phase_a(si0, di0)

            @pl.when(c0 + 4 < n_chunks)
            def _():
                idx_start(c0 + 4, si0, di0, is0, id0)

            phase_w(0)
            red_start(0, red0, rs0)
            phase_r(c0 + 1, 1, red1, rs1, ob1, os1)

        phase_r(n_chunks - 1, 0, red0, rs0, ob0, os0)
        pltpu.make_async_copy(ob1, out_ref(n_chunks - 2), os1).wait()
        pltpu.make_async_copy(ob0, out_ref(n_chunks - 1), os0).wait()

    return k(ht, src, dst)


def kernel(h, edge_index):
    n_nodes, d = h.shape
    n_edges = edge_index.shape[1]
    assert n_edges % (_NC * _C) == 0 and d % (2 * _NS) == 0
    hb = h.astype(jnp.bfloat16)
    ht = jax.lax.bitcast_convert_type(
        hb.reshape(n_nodes, d // 2, 2), jnp.int32).T
    src = edge_index[0].astype(jnp.int32)
    dst = edge_index[1].astype(jnp.int32)
    score = _sc_edge_dot(ht, src, dst, n_edges=n_edges, d=d, n_nodes=n_nodes)
    return score.reshape(n_edges, 1)


# speedup vs baseline: 2.3077x; 1.2726x over previous
import dataclasses
import functools

import jax
import jax.numpy as jnp
from jax import lax
from jax.experimental import pallas as pl
from jax.experimental.pallas import tpu as pltpu
from jax.experimental.pallas import tpu_sc as plsc

_NC = 2
_NS = 16
_L = 16
_C = 1280


@functools.partial(jax.jit, static_argnames=("n_edges", "d", "n_nodes"))
def _sc_edge_dot(ht, src, dst, *, n_edges, d, n_nodes):
    per_sc = n_edges // _NC
    n_chunks = per_sc // _C
    npairs = (n_chunks - 1) // 2
    assert n_chunks == 2 * npairs + 1 and n_chunks >= 3
    nf = d // _NS
    nq = nf // 2
    sub = _C // _NS
    spg = sub // _L
    assert sub % _L == 0

    mesh = plsc.VectorSubcoreMesh(core_axis_name="c", subcore_axis_name="s")
    cp = pltpu.CompilerParams()
    if "needs_layout_passes" in pltpu.CompilerParams.__dataclass_fields__:
        cp = dataclasses.replace(cp, needs_layout_passes=False)

    @functools.partial(
        pl.kernel,
        compiler_params=cp,
        out_type=jax.ShapeDtypeStruct((n_edges,), jnp.float32),
        mesh=mesh,
        scratch_types=[
            pltpu.VMEM((nq, n_nodes), jnp.int32),
            pltpu.VMEM((_C,), jnp.int32),
            pltpu.VMEM((_C,), jnp.int32),
            pltpu.VMEM((_C,), jnp.int32),
            pltpu.VMEM((_C,), jnp.int32),
            pltpu.VMEM((_NS, 128), jnp.float32),
            pltpu.VMEM((_NS, 128), jnp.float32),
            pltpu.VMEM((_NS, 128), jnp.float32),
            pltpu.VMEM((sub,), jnp.float32),
            pltpu.VMEM((sub,), jnp.float32),
            pltpu.VMEM_SHARED((2, _NS, _NS, 128), jnp.float32),
            pltpu.SemaphoreType.DMA,
            pltpu.SemaphoreType.DMA,
            pltpu.SemaphoreType.DMA,
            pltpu.SemaphoreType.DMA,
            pltpu.SemaphoreType.DMA,
            pltpu.SemaphoreType.DMA,
            pltpu.SemaphoreType.DMA,
            pltpu.SemaphoreType.DMA,
            pltpu.SemaphoreType.DMA,
        ],
    )
    def k(ht_hbm, src_hbm, dst_hbm, out_hbm,
          hsl, si0, di0, si1, di1, part_v, red0, red1, ob0, ob1, ex_sh,
          hs_sem, is0, id0, is1, id1, rs0, rs1, os0, os1):
        cid = lax.axis_index("c")
        tid = lax.axis_index("s")
        base_sc = cid * per_sc

        cph = pltpu.make_async_copy(
            ht_hbm.at[pl.ds(tid * nq, nq)], hsl, hs_sem)
        cph.start()

        def idx_start(c, si, di, ssem, dsem):
            pltpu.make_async_copy(
                src_hbm.at[pl.ds(base_sc + c * _C, _C)], si, ssem).start()
            pltpu.make_async_copy(
                dst_hbm.at[pl.ds(base_sc + c * _C, _C)], di, dsem).start()

        def idx_wait(c, si, di, ssem, dsem):
            pltpu.make_async_copy(
                src_hbm.at[pl.ds(base_sc + c * _C, _C)], si, ssem).wait()
            pltpu.make_async_copy(
                dst_hbm.at[pl.ds(base_sc + c * _C, _C)], di, dsem).wait()

        def out_ref(c):
            return out_hbm.at[pl.ds(base_sc + c * _C + tid * sub, sub)]

        def phase_a(si, di):
            @pl.loop(0, _NS)
            def _(p):
                lanes = []
                for j in range(spg):
                    off = p * sub + j * _L
                    lanes.append((j, si[pl.ds(off, _L)], di[pl.ds(off, _L)]))
                for pair in range(0, spg - 1, 2):
                    (jA, sA, dA), (jB, sB, dB) = lanes[pair], lanes[pair + 1]
                    aA0 = aA1 = aB0 = aB1 = jnp.zeros((_L,), jnp.float32)
                    for q in range(nq):
                        row = jnp.full((_L,), q, jnp.int32)
                        wsA = plsc.load_gather(hsl, [row, sA])
                        wsB = plsc.load_gather(hsl, [row, sB])
                        wdA = plsc.load_gather(hsl, [row, dA])
                        wdB = plsc.load_gather(hsl, [row, dB])
                        pmA = (plsc.bitcast(wsA, jnp.bfloat16)
                               * plsc.bitcast(wdA, jnp.bfloat16))
                        pmB = (plsc.bitcast(wsB, jnp.bfloat16)
                               * plsc.bitcast(wdB, jnp.bfloat16))
                        pa, pb = plsc.unpack(
                            pmA, format=plsc.PackFormat.INTERLEAVED,
                            preferred_element_type=jnp.float32)
                        qa, qb = plsc.unpack(
                            pmB, format=plsc.PackFormat.INTERLEAVED,
                            preferred_element_type=jnp.float32)
                        aA0 = aA0 + pa
                        aA1 = aA1 + pb
                        aB0 = aB0 + qa
                        aB1 = aB1 + qb
                    part_v[p, pl.ds(jA * _L, _L)] = aA0 + aA1
                    part_v[p, pl.ds(jB * _L, _L)] = aB0 + aB1
                if spg % 2:
                    jT, sT, dT = lanes[-1]
                    acc0 = acc1 = jnp.zeros((_L,), jnp.float32)
                    for q in range(nq):
                        row = jnp.full((_L,), q, jnp.int32)
                        ws = plsc.load_gather(hsl, [row, sT])
                        wd = plsc.load_gather(hsl, [row, dT])
                        pm = (plsc.bitcast(ws, jnp.bfloat16)
                              * plsc.bitcast(wd, jnp.bfloat16))
                        pa, pb = plsc.unpack(
                            pm, format=plsc.PackFormat.INTERLEAVED,
                            preferred_element_type=jnp.float32)
                        acc0 = acc0 + pa
                        acc1 = acc1 + pb
                    part_v[p, pl.ds(jT * _L, _L)] = acc0 + acc1

        def phase_w(slot):
            pltpu.sync_copy(part_v, ex_sh.at[slot, tid])
            plsc.subcore_barrier()

        def red_start(slot, red, rsem):
            pltpu.make_async_copy(ex_sh.at[slot, :, tid], red, rsem).start()

        def phase_r(c, slot, red, rsem, outb, osem):
            pltpu.make_async_copy(ex_sh.at[slot, :, tid], red, rsem).wait()
            plsc.subcore_barrier()

            @pl.when(c >= 2)
            def _():
                pltpu.make_async_copy(outb, out_ref(c), osem).wait()

            @pl.loop(0, spg)
            def _(j):
                accs = [red[r, pl.ds(j * _L, _L)] for r in range(4)]
                for r in range(4, _NS):
                    accs[r % 4] = accs[r % 4] + red[r, pl.ds(j * _L, _L)]
                outb[pl.ds(j * _L, _L)] = (
                    (accs[0] + accs[1]) + (accs[2] + accs[3]))

            pltpu.make_async_copy(outb, out_ref(c), osem).start()

        idx_start(0, si0, di0, is0, id0)
        idx_start(1, si1, di1, is1, id1)
        cph.wait()

        idx_wait(0, si0, di0, is0, id0)
        phase_a(si0, di0)
        idx_start(2, si0, di0, is0, id0)
        phase_w(0)
        red_start(0, red0, rs0)

        @pl.loop(0, npairs)
        def _(i):
            c0 = 2 * i
            idx_wait(c0 + 1, si1, di1, is1, id1)
            phase_a(si1, di1)

            @pl.when(c0 + 3 < n_chunks)
            def _():
                idx_start(c0 + 3, si1, di1, is1, id1)

            phase_w(1)
            red_start(1, red1, rs1)
            phase_r(c0, 0, red0, rs0, ob0, os0)

            idx_wait(c0 + 2, si0, di0, is0, id0)
            phase_a(si0, di0)

            @pl.when(c0 + 4 < n_chunks)
            def _():
                idx_start(c0 + 4, si0, di0, is0, id0)

            phase_w(0)
            red_start(0, red0, rs0)
            phase_r(c0 + 1, 1, red1, rs1, ob1, os1)

        phase_r(n_chunks - 1, 0, red0, rs0, ob0, os0)
        pltpu.make_async_copy(ob1, out_ref(n_chunks - 2), os1).wait()
        pltpu.make_async_copy(ob0, out_ref(n_chunks - 1), os0).wait()

    return k(ht, src, dst)


def kernel(h, edge_index):
    n_nodes, d = h.shape
    n_edges = edge_index.shape[1]
    assert n_edges % (_NC * _C) == 0 and d % (2 * _NS) == 0
    hb = h.astype(jnp.bfloat16)
    ht = jax.lax.bitcast_convert_type(
        hb.reshape(n_nodes, d // 2, 2), jnp.int32).T
    src = edge_index[0].astype(jnp.int32)
    dst = edge_index[1].astype(jnp.int32)
    score = _sc_edge_dot(ht, src, dst, n_edges=n_edges, d=d, n_nodes=n_nodes)
    return score.reshape(n_edges, 1)


# speedup vs baseline: 2.4971x; 1.0821x over previous
import dataclasses
import functools

import jax
import jax.numpy as jnp
from jax import lax
from jax.experimental import pallas as pl
from jax.experimental.pallas import tpu as pltpu
from jax.experimental.pallas import tpu_sc as plsc

_NC = 2
_NS = 16
_L = 16
_C = 1280


@functools.partial(jax.jit, static_argnames=("n_edges", "d", "n_nodes"))
def _sc_edge_dot(ht, src, dst, *, n_edges, d, n_nodes):
    per_sc = n_edges // _NC
    n_chunks = per_sc // _C
    npairs = (n_chunks - 1) // 2
    assert n_chunks == 2 * npairs + 1 and n_chunks >= 3
    nf = d // _NS
    nq = nf // 2
    sub = _C // _NS
    spg = sub // _L
    assert sub % _L == 0

    mesh = plsc.VectorSubcoreMesh(core_axis_name="c", subcore_axis_name="s")
    cp = pltpu.CompilerParams()
    if "needs_layout_passes" in pltpu.CompilerParams.__dataclass_fields__:
        cp = dataclasses.replace(cp, needs_layout_passes=False)

    @functools.partial(
        pl.kernel,
        compiler_params=cp,
        out_type=jax.ShapeDtypeStruct((n_edges,), jnp.float32),
        mesh=mesh,
        scratch_types=[
            pltpu.VMEM((nq, n_nodes), jnp.int32),
            pltpu.VMEM((_C,), jnp.int32),
            pltpu.VMEM((_C,), jnp.int32),
            pltpu.VMEM((_C,), jnp.int32),
            pltpu.VMEM((_C,), jnp.int32),
            pltpu.VMEM((_NS, 128), jnp.float32),
            pltpu.VMEM((_NS, 128), jnp.float32),
            pltpu.VMEM((_NS, 128), jnp.float32),
            pltpu.VMEM((sub,), jnp.float32),
            pltpu.VMEM((sub,), jnp.float32),
            pltpu.VMEM_SHARED((2, _NS, _NS, 128), jnp.float32),
            pltpu.SemaphoreType.DMA,
            pltpu.SemaphoreType.DMA,
            pltpu.SemaphoreType.DMA,
            pltpu.SemaphoreType.DMA,
            pltpu.SemaphoreType.DMA,
            pltpu.SemaphoreType.DMA,
            pltpu.SemaphoreType.DMA,
            pltpu.SemaphoreType.DMA,
            pltpu.SemaphoreType.DMA,
        ],
    )
    def k(ht_hbm, src_hbm, dst_hbm, out_hbm,
          hsl, si0, di0, si1, di1, part_v, red0, red1, ob0, ob1, ex_sh,
          hs_sem, is0, id0, is1, id1, rs0, rs1, os0, os1):
        cid = lax.axis_index("c")
        tid = lax.axis_index("s")
        base_sc = cid * per_sc

        cph = pltpu.make_async_copy(
            ht_hbm.at[pl.ds(tid * nq, nq)], hsl, hs_sem)
        cph.start()

        def idx_start(c, si, di, ssem, dsem):
            pltpu.make_async_copy(
                src_hbm.at[pl.ds(base_sc + c * _C, _C)], si, ssem).start()
            pltpu.make_async_copy(
                dst_hbm.at[pl.ds(base_sc + c * _C, _C)], di, dsem).start()

        def idx_wait(c, si, di, ssem, dsem):
            pltpu.make_async_copy(
                src_hbm.at[pl.ds(base_sc + c * _C, _C)], si, ssem).wait()
            pltpu.make_async_copy(
                dst_hbm.at[pl.ds(base_sc + c * _C, _C)], di, dsem).wait()

        def out_ref(c):
            return out_hbm.at[pl.ds(base_sc + c * _C + tid * sub, sub)]

        def phase_a(si, di):
            @pl.loop(0, _NS)
            def _(p):
                lanes = []
                for j in range(spg):
                    off = p * sub + j * _L
                    lanes.append((j, si[pl.ds(off, _L)], di[pl.ds(off, _L)]))
                nway = 4
                assert spg % nway in (0, 1)
                for base in range(0, spg - (spg % nway), nway):
                    grp = lanes[base:base + nway]
                    a0 = [jnp.zeros((_L,), jnp.float32) for _ in grp]
                    a1 = [jnp.zeros((_L,), jnp.float32) for _ in grp]
                    for q in range(nq):
                        row = jnp.full((_L,), q, jnp.int32)
                        wss = [plsc.load_gather(hsl, [row, s])
                               for (_, s, _d) in grp]
                        wds = [plsc.load_gather(hsl, [row, _d])
                               for (_, _s, _d) in grp]
                        pms = [plsc.bitcast(ws, jnp.bfloat16)
                               * plsc.bitcast(wd, jnp.bfloat16)
                               for ws, wd in zip(wss, wds)]
                        for gi, pm in enumerate(pms):
                            pa, pb = plsc.unpack(
                                pm, format=plsc.PackFormat.INTERLEAVED,
                                preferred_element_type=jnp.float32)
                            a0[gi] = a0[gi] + pa
                            a1[gi] = a1[gi] + pb
                    for gi, (jG, _s, _d) in enumerate(grp):
                        part_v[p, pl.ds(jG * _L, _L)] = a0[gi] + a1[gi]
                if spg % nway:
                    jT, sT, dT = lanes[-1]
                    acc0 = acc1 = jnp.zeros((_L,), jnp.float32)
                    for q in range(nq):
                        row = jnp.full((_L,), q, jnp.int32)
                        ws = plsc.load_gather(hsl, [row, sT])
                        wd = plsc.load_gather(hsl, [row, dT])
                        pm = (plsc.bitcast(ws, jnp.bfloat16)
                              * plsc.bitcast(wd, jnp.bfloat16))
                        pa, pb = plsc.unpack(
                            pm, format=plsc.PackFormat.INTERLEAVED,
                            preferred_element_type=jnp.float32)
                        acc0 = acc0 + pa
                        acc1 = acc1 + pb
                    part_v[p, pl.ds(jT * _L, _L)] = acc0 + acc1

        def phase_w(slot):
            pltpu.sync_copy(part_v, ex_sh.at[slot, tid])
            plsc.subcore_barrier()

        def red_start(slot, red, rsem):
            pltpu.make_async_copy(ex_sh.at[slot, :, tid], red, rsem).start()

        def phase_r(c, slot, red, rsem, outb, osem):
            pltpu.make_async_copy(ex_sh.at[slot, :, tid], red, rsem).wait()
            plsc.subcore_barrier()

            @pl.when(c >= 2)
            def _():
                pltpu.make_async_copy(outb, out_ref(c), osem).wait()

            @pl.loop(0, spg)
            def _(j):
                accs = [red[r, pl.ds(j * _L, _L)] for r in range(4)]
                for r in range(4, _NS):
                    accs[r % 4] = accs[r % 4] + red[r, pl.ds(j * _L, _L)]
                outb[pl.ds(j * _L, _L)] = (
                    (accs[0] + accs[1]) + (accs[2] + accs[3]))

            pltpu.make_async_copy(outb, out_ref(c), osem).start()

        idx_start(0, si0, di0, is0, id0)
        idx_start(1, si1, di1, is1, id1)
        cph.wait()

        idx_wait(0, si0, di0, is0, id0)
        phase_a(si0, di0)
        idx_start(2, si0, di0, is0, id0)
        phase_w(0)
        red_start(0, red0, rs0)

        @pl.loop(0, npairs)
        def _(i):
            c0 = 2 * i
            idx_wait(c0 + 1, si1, di1, is1, id1)
            phase_a(si1, di1)

            @pl.when(c0 + 3 < n_chunks)
            def _():
                idx_start(c0 + 3, si1, di1, is1, id1)

            phase_w(1)
            red_start(1, red1, rs1)
            phase_r(c0, 0, red0, rs0, ob0, os0)

            idx_wait(c0 + 2, si0, di0, is0, id0)
            phase_a(si0, di0)

            @pl.when(c0 + 4 < n_chunks)
            def _():
                idx_start(c0 + 4, si0, di0, is0, id0)

            phase_w(0)
            red_start(0, red0, rs0)
            phase_r(c0 + 1, 1, red1, rs1, ob1, os1)

        phase_r(n_chunks - 1, 0, red0, rs0, ob0, os0)
        pltpu.make_async_copy(ob1, out_ref(n_chunks - 2), os1).wait()
        pltpu.make_async_copy(ob0, out_ref(n_chunks - 1), os0).wait()

    return k(ht, src, dst)


def kernel(h, edge_index):
    n_nodes, d = h.shape
    n_edges = edge_index.shape[1]
    assert n_edges % (_NC * _C) == 0 and d % (2 * _NS) == 0
    hb = h.astype(jnp.bfloat16)
    ht = jax.lax.bitcast_convert_type(
        hb.reshape(n_nodes, d // 2, 2), jnp.int32).T
    src = edge_index[0].astype(jnp.int32)
    dst = edge_index[1].astype(jnp.int32)
    score = _sc_edge_dot(ht, src, dst, n_edges=n_edges, d=d, n_nodes=n_nodes)
    return score.reshape(n_edges, 1)


# speedup vs baseline: 2.6752x; 1.0713x over previous
import dataclasses
import functools

import jax
import jax.numpy as jnp
from jax import lax
from jax.experimental import pallas as pl
from jax.experimental.pallas import tpu as pltpu
from jax.experimental.pallas import tpu_sc as plsc

_NC = 2
_NS = 16
_L = 16
_C = 1280


@functools.partial(jax.jit, static_argnames=("n_edges", "d", "n_nodes"))
def _sc_edge_dot(ht, src, dst, *, n_edges, d, n_nodes):
    per_sc = n_edges // _NC
    n_chunks = per_sc // _C
    npairs = (n_chunks - 1) // 2
    assert n_chunks == 2 * npairs + 1 and n_chunks >= 3
    nf = d // _NS
    nq = nf // 2
    sub = _C // _NS
    spg = sub // _L
    assert sub % _L == 0

    mesh = plsc.VectorSubcoreMesh(core_axis_name="c", subcore_axis_name="s")
    cp = pltpu.CompilerParams()
    if "needs_layout_passes" in pltpu.CompilerParams.__dataclass_fields__:
        cp = dataclasses.replace(cp, needs_layout_passes=False)

    @functools.partial(
        pl.kernel,
        compiler_params=cp,
        out_type=jax.ShapeDtypeStruct((n_edges,), jnp.float32),
        mesh=mesh,
        scratch_types=[
            pltpu.VMEM((nq, n_nodes), jnp.int32),
            pltpu.VMEM((_C,), jnp.int32),
            pltpu.VMEM((_C,), jnp.int32),
            pltpu.VMEM((_C,), jnp.int32),
            pltpu.VMEM((_C,), jnp.int32),
            pltpu.VMEM((_NS, 128), jnp.float32),
            pltpu.VMEM((_NS, 128), jnp.float32),
            pltpu.VMEM((_NS, 128), jnp.float32),
            pltpu.VMEM((sub,), jnp.float32),
            pltpu.VMEM((sub,), jnp.float32),
            pltpu.VMEM_SHARED((2, _NS, _NS, 128), jnp.float32),
            pltpu.SemaphoreType.DMA,
            pltpu.SemaphoreType.DMA,
            pltpu.SemaphoreType.DMA,
            pltpu.SemaphoreType.DMA,
            pltpu.SemaphoreType.DMA,
            pltpu.SemaphoreType.DMA,
            pltpu.SemaphoreType.DMA,
            pltpu.SemaphoreType.DMA,
            pltpu.SemaphoreType.DMA,
        ],
    )
    def k(ht_hbm, src_hbm, dst_hbm, out_hbm,
          hsl, si0, di0, si1, di1, part_v, red0, red1, ob0, ob1, ex_sh,
          hs_sem, is0, id0, is1, id1, rs0, rs1, os0, os1):
        cid = lax.axis_index("c")
        tid = lax.axis_index("s")
        base_sc = cid * per_sc

        cph = pltpu.make_async_copy(
            ht_hbm.at[pl.ds(tid * nq, nq)], hsl, hs_sem)
        cph.start()

        def idx_start(c, si, di, ssem, dsem):
            pltpu.make_async_copy(
                src_hbm.at[pl.ds(base_sc + c * _C, _C)], si, ssem).start()
            pltpu.make_async_copy(
                dst_hbm.at[pl.ds(base_sc + c * _C, _C)], di, dsem).start()

        def idx_wait(c, si, di, ssem, dsem):
            pltpu.make_async_copy(
                src_hbm.at[pl.ds(base_sc + c * _C, _C)], si, ssem).wait()
            pltpu.make_async_copy(
                dst_hbm.at[pl.ds(base_sc + c * _C, _C)], di, dsem).wait()

        def out_ref(c):
            return out_hbm.at[pl.ds(base_sc + c * _C + tid * sub, sub)]

        def phase_a(si, di):
            @pl.loop(0, _NS)
            def _(p):
                lanes = []
                for j in range(spg):
                    off = p * sub + j * _L
                    lanes.append((j, si[pl.ds(off, _L)], di[pl.ds(off, _L)]))
                nway = 5
                assert spg % nway in (0, 1)
                for base in range(0, spg - (spg % nway), nway):
                    grp = lanes[base:base + nway]
                    a0 = [jnp.zeros((_L,), jnp.float32) for _ in grp]
                    a1 = [jnp.zeros((_L,), jnp.float32) for _ in grp]
                    for q in range(nq):
                        row = jnp.full((_L,), q, jnp.int32)
                        wss = [plsc.load_gather(hsl, [row, s])
                               for (_, s, _d) in grp]
                        wds = [plsc.load_gather(hsl, [row, _d])
                               for (_, _s, _d) in grp]
                        pms = [plsc.bitcast(ws, jnp.bfloat16)
                               * plsc.bitcast(wd, jnp.bfloat16)
                               for ws, wd in zip(wss, wds)]
                        for gi, pm in enumerate(pms):
                            pa, pb = plsc.unpack(
                                pm, format=plsc.PackFormat.INTERLEAVED,
                                preferred_element_type=jnp.float32)
                            a0[gi] = a0[gi] + pa
                            a1[gi] = a1[gi] + pb
                    for gi, (jG, _s, _d) in enumerate(grp):
                        part_v[p, pl.ds(jG * _L, _L)] = a0[gi] + a1[gi]
                if spg % nway:
                    jT, sT, dT = lanes[-1]
                    acc0 = acc1 = jnp.zeros((_L,), jnp.float32)
                    for q in range(nq):
                        row = jnp.full((_L,), q, jnp.int32)
                        ws = plsc.load_gather(hsl, [row, sT])
                        wd = plsc.load_gather(hsl, [row, dT])
                        pm = (plsc.bitcast(ws, jnp.bfloat16)
                              * plsc.bitcast(wd, jnp.bfloat16))
                        pa, pb = plsc.unpack(
                            pm, format=plsc.PackFormat.INTERLEAVED,
                            preferred_element_type=jnp.float32)
                        acc0 = acc0 + pa
                        acc1 = acc1 + pb
                    part_v[p, pl.ds(jT * _L, _L)] = acc0 + acc1

        def phase_w(slot):
            pltpu.sync_copy(part_v, ex_sh.at[slot, tid])
            plsc.subcore_barrier()

        def red_start(slot, red, rsem):
            pltpu.make_async_copy(ex_sh.at[slot, :, tid], red, rsem).start()

        def phase_r(c, slot, red, rsem, outb, osem):
            pltpu.make_async_copy(ex_sh.at[slot, :, tid], red, rsem).wait()
            plsc.subcore_barrier()

            @pl.when(c >= 2)
            def _():
                pltpu.make_async_copy(outb, out_ref(c), osem).wait()

            @pl.loop(0, spg)
            def _(j):
                accs = [red[r, pl.ds(j * _L, _L)] for r in range(4)]
                for r in range(4, _NS):
                    accs[r % 4] = accs[r % 4] + red[r, pl.ds(j * _L, _L)]
                outb[pl.ds(j * _L, _L)] = (
                    (accs[0] + accs[1]) + (accs[2] + accs[3]))

            pltpu.make_async_copy(outb, out_ref(c), osem).start()

        idx_start(0, si0, di0, is0, id0)
        idx_start(1, si1, di1, is1, id1)
        cph.wait()

        idx_wait(0, si0, di0, is0, id0)
        phase_a(si0, di0)
        idx_start(2, si0, di0, is0, id0)
        phase_w(0)
        red_start(0, red0, rs0)

        @pl.loop(0, npairs)
        def _(i):
            c0 = 2 * i
            idx_wait(c0 + 1, si1, di1, is1, id1)
            phase_a(si1, di1)

            @pl.when(c0 + 3 < n_chunks)
            def _():
                idx_start(c0 + 3, si1, di1, is1, id1)

            phase_w(1)
            red_start(1, red1, rs1)
            phase_r(c0, 0, red0, rs0, ob0, os0)

            idx_wait(c0 + 2, si0, di0, is0, id0)
            phase_a(si0, di0)

            @pl.when(c0 + 4 < n_chunks)
            def _():
                idx_start(c0 + 4, si0, di0, is0, id0)

            phase_w(0)
            red_start(0, red0, rs0)
            phase_r(c0 + 1, 1, red1, rs1, ob1, os1)

        phase_r(n_chunks - 1, 0, red0, rs0, ob0, os0)
        pltpu.make_async_copy(ob1, out_ref(n_chunks - 2), os1).wait()
        pltpu.make_async_copy(ob0, out_ref(n_chunks - 1), os0).wait()

    return k(ht, src, dst)


def kernel(h, edge_index):
    n_nodes, d = h.shape
    n_edges = edge_index.shape[1]
    assert n_edges % (_NC * _C) == 0 and d % (2 * _NS) == 0
    hb = h.astype(jnp.bfloat16)
    ht = jax.lax.bitcast_convert_type(
        hb.reshape(n_nodes, d // 2, 2), jnp.int32).T
    src = edge_index[0].astype(jnp.int32)
    dst = edge_index[1].astype(jnp.int32)
    score = _sc_edge_dot(ht, src, dst, n_edges=n_edges, d=d, n_nodes=n_nodes)
    return score.reshape(n_edges, 1)


# speedup vs baseline: 2.7374x; 1.0233x over previous
import dataclasses
import functools

import jax
import jax.numpy as jnp
from jax import lax
from jax.experimental import pallas as pl
from jax.experimental.pallas import tpu as pltpu
from jax.experimental.pallas import tpu_sc as plsc

_NC = 2
_NS = 16
_L = 16
_C = 1280


@functools.partial(jax.jit, static_argnames=("n_edges", "d", "n_nodes"))
def _sc_edge_dot(ht, src, dst, *, n_edges, d, n_nodes):
    per_sc = n_edges // _NC
    n_chunks = per_sc // _C
    npairs = (n_chunks - 1) // 2
    assert n_chunks == 2 * npairs + 1 and n_chunks >= 3
    nf = d // _NS
    nq = nf // 2
    sub = _C // _NS
    spg = sub // _L
    assert sub % _L == 0 and nq == 4

    mesh = plsc.VectorSubcoreMesh(core_axis_name="c", subcore_axis_name="s")
    cp = pltpu.CompilerParams()
    if "needs_layout_passes" in pltpu.CompilerParams.__dataclass_fields__:
        cp = dataclasses.replace(cp, needs_layout_passes=False)

    @functools.partial(
        pl.kernel,
        compiler_params=cp,
        out_type=jax.ShapeDtypeStruct((n_edges,), jnp.float32),
        mesh=mesh,
        scratch_types=[
            pltpu.VMEM((nq, n_nodes), jnp.int32),
            pltpu.VMEM((_C,), jnp.int32),
            pltpu.VMEM((_C,), jnp.int32),
            pltpu.VMEM((_C,), jnp.int32),
            pltpu.VMEM((_C,), jnp.int32),
            pltpu.VMEM((_NS, 128), jnp.float32),
            pltpu.VMEM((_NS, 128), jnp.float32),
            pltpu.VMEM((_NS, 128), jnp.float32),
            pltpu.VMEM((sub,), jnp.float32),
            pltpu.VMEM((sub,), jnp.float32),
            pltpu.VMEM_SHARED((2, _NS, _NS, 128), jnp.float32),
            pltpu.SemaphoreType.DMA,
            pltpu.SemaphoreType.DMA,
            pltpu.SemaphoreType.DMA,
            pltpu.SemaphoreType.DMA,
            pltpu.SemaphoreType.DMA,
            pltpu.SemaphoreType.DMA,
            pltpu.SemaphoreType.DMA,
            pltpu.SemaphoreType.DMA,
            pltpu.SemaphoreType.DMA,
        ],
    )
    def k(ht_hbm, src_hbm, dst_hbm, out_hbm,
          hsl, si0, di0, si1, di1, part_v, red0, red1, ob0, ob1, ex_sh,
          hs_sem, is0, id0, is1, id1, rs0, rs1, os0, os1):
        cid = lax.axis_index("c")
        tid = lax.axis_index("s")
        base_sc = cid * per_sc

        cph = pltpu.make_async_copy(
            ht_hbm.at[pl.ds(tid * nq, nq)], hsl, hs_sem)
        cph.start()

        def idx_start(c, si, di, ssem, dsem):
            pltpu.make_async_copy(
                src_hbm.at[pl.ds(base_sc + c * _C, _C)], si, ssem).start()
            pltpu.make_async_copy(
                dst_hbm.at[pl.ds(base_sc + c * _C, _C)], di, dsem).start()

        def idx_wait(c, si, di, ssem, dsem):
            pltpu.make_async_copy(
                src_hbm.at[pl.ds(base_sc + c * _C, _C)], si, ssem).wait()
            pltpu.make_async_copy(
                dst_hbm.at[pl.ds(base_sc + c * _C, _C)], di, dsem).wait()

        def out_ref(c):
            return out_hbm.at[pl.ds(base_sc + c * _C + tid * sub, sub)]

        def phase_a(si, di):
            @pl.loop(0, _NS)
            def _(p):
                lanes = []
                for j in range(spg):
                    off = p * sub + j * _L
                    lanes.append((j, si[pl.ds(off, _L)], di[pl.ds(off, _L)]))
                nway = 5
                assert spg % nway in (0, 1)
                for base in range(0, spg - (spg % nway), nway):
                    grp = lanes[base:base + nway]
                    pp = [[None] * nq for _ in grp]
                    for q in range(nq):
                        row = jnp.full((_L,), q, jnp.int32)
                        wss = [plsc.load_gather(hsl, [row, s])
                               for (_, s, _d) in grp]
                        wds = [plsc.load_gather(hsl, [row, _d])
                               for (_, _s, _d) in grp]
                        for gi, (ws, wd) in enumerate(zip(wss, wds)):
                            pp[gi][q] = (plsc.bitcast(ws, jnp.bfloat16)
                                         * plsc.bitcast(wd, jnp.bfloat16))
                    for gi, (jG, _s, _d) in enumerate(grp):
                        accp = pp[gi][0] + pp[gi][1]
                        accq = pp[gi][2] + pp[gi][3]
                        pa, pb = plsc.unpack(
                            accp, format=plsc.PackFormat.INTERLEAVED,
                            preferred_element_type=jnp.float32)
                        qa, qb = plsc.unpack(
                            accq, format=plsc.PackFormat.INTERLEAVED,
                            preferred_element_type=jnp.float32)
                        part_v[p, pl.ds(jG * _L, _L)] = (
                            (pa + pb) + (qa + qb))
                if spg % nway:
                    jT, sT, dT = lanes[-1]
                    acc0 = acc1 = jnp.zeros((_L,), jnp.float32)
                    for q in range(nq):
                        row = jnp.full((_L,), q, jnp.int32)
                        ws = plsc.load_gather(hsl, [row, sT])
                        wd = plsc.load_gather(hsl, [row, dT])
                        pm = (plsc.bitcast(ws, jnp.bfloat16)
                              * plsc.bitcast(wd, jnp.bfloat16))
                        pa, pb = plsc.unpack(
                            pm, format=plsc.PackFormat.INTERLEAVED,
                            preferred_element_type=jnp.float32)
                        acc0 = acc0 + pa
                        acc1 = acc1 + pb
                    part_v[p, pl.ds(jT * _L, _L)] = acc0 + acc1

        def phase_w(slot):
            pltpu.sync_copy(part_v, ex_sh.at[slot, tid])
            plsc.subcore_barrier()

        def red_start(slot, red, rsem):
            pltpu.make_async_copy(ex_sh.at[slot, :, tid], red, rsem).start()

        def phase_r(c, slot, red, rsem, outb, osem):
            pltpu.make_async_copy(ex_sh.at[slot, :, tid], red, rsem).wait()
            plsc.subcore_barrier()

            @pl.when(c >= 2)
            def _():
                pltpu.make_async_copy(outb, out_ref(c), osem).wait()

            @pl.loop(0, spg)
            def _(j):
                accs = [red[r, pl.ds(j * _L, _L)] for r in range(4)]
                for r in range(4, _NS):
                    accs[r % 4] = accs[r % 4] + red[r, pl.ds(j * _L, _L)]
                outb[pl.ds(j * _L, _L)] = (
                    (accs[0] + accs[1]) + (accs[2] + accs[3]))

            pltpu.make_async_copy(outb, out_ref(c), osem).start()

        idx_start(0, si0, di0, is0, id0)
        idx_start(1, si1, di1, is1, id1)
        cph.wait()

        idx_wait(0, si0, di0, is0, id0)
        phase_a(si0, di0)
        idx_start(2, si0, di0, is0, id0)
        phase_w(0)
        red_start(0, red0, rs0)

        @pl.loop(0, npairs)
        def _(i):
            c0 = 2 * i
            idx_wait(c0 + 1, si1, di1, is1, id1)
            phase_a(si1, di1)

            @pl.when(c0 + 3 < n_chunks)
            def _():
                idx_start(c0 + 3, si1, di1, is1, id1)

            phase_w(1)
            red_start(1, red1, rs1)
            phase_r(c0, 0, red0, rs0, ob0, os0)

            idx_wait(c0 + 2, si0, di0, is0, id0)
            phase_a(si0, di0)

            @pl.when(c0 + 4 < n_chunks)
            def _():
                idx_start(c0 + 4, si0, di0, is0, id0)

            phase_w(0)
            red_start(0, red0, rs0)
            phase_r(c0 + 1, 1, red1, rs1, ob1, os1)

        phase_r(n_chunks - 1, 0, red0, rs0, ob0, os0)
        pltpu.make_async_copy(ob1, out_ref(n_chunks - 2), os1).wait()
        pltpu.make_async_copy(ob0, out_ref(n_chunks - 1), os0).wait()

    return k(ht, src, dst)


def kernel(h, edge_index):
    n_nodes, d = h.shape
    n_edges = edge_index.shape[1]
    assert n_edges % (_NC * _C) == 0 and d % (2 * _NS) == 0
    hb = h.astype(jnp.bfloat16)
    ht = jax.lax.bitcast_convert_type(
        hb.reshape(n_nodes, d // 2, 2), jnp.int32).T
    src = edge_index[0].astype(jnp.int32)
    dst = edge_index[1].astype(jnp.int32)
    score = _sc_edge_dot(ht, src, dst, n_edges=n_edges, d=d, n_nodes=n_nodes)
    return score.reshape(n_edges, 1)
